# baseline (speedup 1.0000x reference)
import functools

import jax
import jax.numpy as jnp
from jax import lax
from jax.experimental import pallas as pl
from jax.experimental.pallas import tpu as pltpu
from jax.experimental.pallas import tpu_sc as plsc

_NC = 2
_NS = 16
_NW = _NC * _NS
_LANES = 16

_RND = 12582912.0


def _make_sc_lookup(nrows, seq, seq_pad, D, ch):
    rows_per_w = nrows // _NW
    per = rows_per_w * seq
    nch = per // ch
    nfull = seq // _LANES
    tail = seq - nfull * _LANES
    B = nrows * seq
    mesh = plsc.VectorSubcoreMesh(core_axis_name="c", subcore_axis_name="s")

    @functools.partial(
        pl.kernel,
        mesh=mesh,
        out_type=jax.ShapeDtypeStruct((B, D), jnp.float32),
        scratch_types=[
            pltpu.VMEM((rows_per_w, seq_pad), jnp.float32),
            pltpu.VMEM((per + _LANES,), jnp.int32),
            pltpu.VMEM((2, ch, D), jnp.float32),
            pltpu.SemaphoreType.DMA,
            pltpu.SemaphoreType.DMA,
            pltpu.SemaphoreType.DMA,
        ],
        compiler_params=pltpu.CompilerParams(use_tc_tiling_on_sc=False),
    )
    def run(x_hbm, tab_hbm, out_hbm, x_v, idx_v, rows_v, gsem, wsem0, wsem1):
        wid = lax.axis_index("s") * _NC + lax.axis_index("c")
        row0 = wid * rows_per_w
        base = row0 * seq
        pltpu.sync_copy(x_hbm.at[pl.ds(row0, rows_per_w)], x_v)

        def row_body(r, carry):
            def tok_body(g, c2):
                xv = x_v[r, pl.ds(g * _LANES, _LANES)]
                v = jnp.minimum(jnp.maximum(xv, -65.0), 65.0)
                rr = (v + _RND) - _RND
                t = rr.astype(jnp.int32)
                t = jnp.minimum(jnp.maximum(t, -64), 64) + 65
                t = jnp.where(xv == jnp.inf, 130, t)
                t = jnp.where(xv == -jnp.inf, 0, t)
                t = jnp.where(xv != xv, 0, t)
                idx_v[pl.ds(r * seq + g * _LANES, _LANES)] = t
                return c2

            ngroups = nfull + (1 if tail else 0)
            lax.fori_loop(0, ngroups, tok_body, carry)
            return carry

        lax.fori_loop(0, rows_per_w, row_body, 0)

        writes = [None, None]
        for c in range(nch):
            b = c % 2
            wsem = wsem0 if b == 0 else wsem1
            if writes[b] is not None:
                writes[b].wait()
            gathers = [
                pltpu.async_copy(
                    tab_hbm.at[idx_v.at[pl.ds(c * ch + j * 128, 128)]],
                    rows_v.at[b, pl.ds(j * 128, 128)],
                    gsem,
                )
                for j in range(ch // 128)
            ]
            for cp in gathers:
                cp.wait()
            writes[b] = pltpu.async_copy(
                rows_v.at[b], out_hbm.at[pl.ds(base + c * ch, ch)], wsem
            )
        for w in writes:
            if w is not None:
                w.wait()

    return run


def kernel(x, table):
    b, seq = x.shape[0], x.shape[1]
    D = table.shape[1]
    xs = jnp.squeeze(x, -1)
    seq_pad = 256
    xp = jnp.pad(xs, ((0, 0), (0, seq_pad - seq)))
    out = _make_sc_lookup(b, seq, seq_pad, D, ch=512)(xp, table)
    return out.reshape(b, seq, D)
